# baseline (speedup 1.0000x reference)
import functools

import jax
import jax.numpy as jnp
from jax import lax
from jax.experimental import pallas as pl
from jax.experimental.pallas import tpu as pltpu
from jax.experimental.pallas import tpu_sc as plsc

_B = 16384
_E = 97
_EP = 112
_D = 199
_C = 20
_BS = 8192

_NC = 2
_NSUB = 16
_NW = _NC * _NSUB
_BPW = _B // _NW


def _tc_body(x_ref, wt_ref, lin_ref, idx_ref):
    xb = x_ref[...]
    lin_ref[...] = jax.lax.dot_general(
        xb, wt_ref[...], (((1,), (0,)), ((), ())),
        preferred_element_type=jnp.float32)
    idx_ref[...] = xb[:, 0:1].astype(jnp.int32)


def _tc_stage(x, wt_pad):
    return pl.pallas_call(
        _tc_body,
        grid=(_B // _BS,),
        in_specs=[
            pl.BlockSpec((_BS, _D + 1), lambda i: (i, 0)),
            pl.BlockSpec((_D + 1, _EP), lambda i: (0, 0)),
        ],
        out_specs=[
            pl.BlockSpec((_BS, _EP), lambda i: (i, 0)),
            pl.BlockSpec((_BS, 1), lambda i: (i, 0)),
        ],
        out_shape=[
            jax.ShapeDtypeStruct((_B, _EP), jnp.float32),
            jax.ShapeDtypeStruct((_B, 1), jnp.int32),
        ],
    )(x, wt_pad)


@functools.partial(
    pl.kernel,
    mesh=plsc.VectorSubcoreMesh(core_axis_name="c", subcore_axis_name="s"),
    compiler_params=pltpu.CompilerParams(use_tc_tiling_on_sc=False),
    out_type=jax.ShapeDtypeStruct((_B, _E), jnp.float32),
    scratch_types=[
        pltpu.VMEM((_BPW,), jnp.int32),
        pltpu.VMEM((_BPW, _EP), jnp.float32),
        pltpu.VMEM((_BPW, _EP), jnp.float32),
        pltpu.SemaphoreType.DMA,
    ],
)
def _sc_gather_add(lin_hbm, idx_hbm, table_hbm, out_hbm,
                   idx_v, rows_v, lin_v, sem):
    wid = lax.axis_index("s") * _NC + lax.axis_index("c")
    base = wid * _BPW
    pltpu.sync_copy(idx_hbm.at[pl.ds(base, _BPW)], idx_v)
    gat = pltpu.async_copy(table_hbm.at[idx_v], rows_v, sem)
    pltpu.sync_copy(lin_hbm.at[pl.ds(base, _BPW)], lin_v)
    gat.wait()

    def row_body(r, carry):
        for c in range(0, _EP, 16):
            plsc.addupdate(rows_v.at[r, pl.ds(c, 16)],
                           lin_v[r, pl.ds(c, 16)])
        return carry

    lax.fori_loop(0, _BPW, row_body, 0)

    for c in range(0, 96, 16):
        pltpu.sync_copy(rows_v.at[:, pl.ds(c, 16)],
                        out_hbm.at[pl.ds(base, _BPW), pl.ds(c, 16)])
    pltpu.sync_copy(rows_v.at[:, pl.ds(96, 1)],
                    out_hbm.at[pl.ds(base, _BPW), pl.ds(96, 1)])


def kernel(x, emb_table, W, b):
    wt_pad = jnp.zeros((_D + 1, _EP), jnp.float32)
    wt_pad = wt_pad.at[1:, :_E].set(W.T)
    table2 = jnp.zeros((_C, _EP), jnp.float32)
    table2 = table2.at[:, :_E].set(emb_table + b.reshape(1, _E))
    lin, idx2 = _tc_stage(x, wt_pad)
    idx = idx2.reshape(_B)
    return _sc_gather_add(lin, idx, table2)


# speedup vs baseline: 1.2517x; 1.2517x over previous
import functools

import jax
import jax.numpy as jnp
from jax import lax
from jax.experimental import pallas as pl
from jax.experimental.pallas import tpu as pltpu
from jax.experimental.pallas import tpu_sc as plsc

_B = 16384
_E = 97
_EP = 112
_D = 199
_C = 20
_BS = 8192

_NC = 2
_NSUB = 16
_NW = _NC * _NSUB
_BPW = _B // _NW


def _tc_body(x_ref, wt_ref, lin_ref, idx_ref):
    xb = x_ref[...]
    lin_ref[...] = jax.lax.dot_general(
        xb, wt_ref[...], (((1,), (0,)), ((), ())),
        preferred_element_type=jnp.float32)
    idx_ref[...] = xb[:, 0:1].astype(jnp.int32)


def _tc_stage(x, wt_pad):
    return pl.pallas_call(
        _tc_body,
        grid=(_B // _BS,),
        in_specs=[
            pl.BlockSpec((_BS, _D + 1), lambda i: (i, 0)),
            pl.BlockSpec((_D + 1, _EP), lambda i: (0, 0)),
        ],
        out_specs=[
            pl.BlockSpec((_BS, _EP), lambda i: (i, 0)),
            pl.BlockSpec((_BS, 1), lambda i: (i, 0)),
        ],
        out_shape=[
            jax.ShapeDtypeStruct((_B, _EP), jnp.float32),
            jax.ShapeDtypeStruct((_B, 1), jnp.int32),
        ],
    )(x, wt_pad)


@functools.partial(
    pl.kernel,
    mesh=plsc.VectorSubcoreMesh(core_axis_name="c", subcore_axis_name="s"),
    compiler_params=pltpu.CompilerParams(use_tc_tiling_on_sc=False),
    out_type=jax.ShapeDtypeStruct((_B, _E), jnp.float32),
    scratch_types=[
        pltpu.VMEM((_BPW,), jnp.int32),
        pltpu.VMEM((_BPW, _EP), jnp.float32),
        pltpu.VMEM((_BPW, _EP), jnp.float32),
        pltpu.VMEM_SHARED((_C, _EP), jnp.float32),
        pltpu.SemaphoreType.DMA,
        pltpu.SemaphoreType.DMA,
    ],
)
def _sc_gather_add(lin_hbm, idx_hbm, table_hbm, out_hbm,
                   idx_v, rows_v, lin_v, table_sp, sem, sem2):
    wid = lax.axis_index("s") * _NC + lax.axis_index("c")
    base = wid * _BPW
    pltpu.sync_copy(idx_hbm.at[pl.ds(base, _BPW)], idx_v)

    @pl.when(lax.axis_index("s") == 0)
    def _():
        pltpu.sync_copy(table_hbm, table_sp)

    plsc.subcore_barrier()

    gat = pltpu.async_copy(table_sp.at[idx_v], rows_v, sem)
    pltpu.sync_copy(lin_hbm.at[pl.ds(base, _BPW)], lin_v)
    gat.wait()

    def row_body(r, carry):
        for c in range(0, _EP, 16):
            plsc.addupdate(rows_v.at[r, pl.ds(c, 16)],
                           lin_v[r, pl.ds(c, 16)])
        return carry

    lax.fori_loop(0, _BPW, row_body, 0)

    copies = []
    for c in range(0, 96, 16):
        copies.append(pltpu.async_copy(
            rows_v.at[:, pl.ds(c, 16)],
            out_hbm.at[pl.ds(base, _BPW), pl.ds(c, 16)], sem2))
    copies.append(pltpu.async_copy(
        rows_v.at[:, pl.ds(96, 1)],
        out_hbm.at[pl.ds(base, _BPW), pl.ds(96, 1)], sem2))
    for cp in copies:
        cp.wait()


def kernel(x, emb_table, W, b):
    wt_pad = jnp.zeros((_D + 1, _EP), jnp.float32)
    wt_pad = wt_pad.at[1:, :_E].set(W.T)
    table2 = jnp.zeros((_C, _EP), jnp.float32)
    table2 = table2.at[:, :_E].set(emb_table + b.reshape(1, _E))
    lin, idx2 = _tc_stage(x, wt_pad)
    idx = idx2.reshape(_B)
    return _sc_gather_add(lin, idx, table2)


# speedup vs baseline: 4.2941x; 3.4305x over previous
import jax
import jax.numpy as jnp
from jax.experimental import pallas as pl

_B = 16384
_E = 97
_D = 199
_C = 20
_BS = 8192


def _body(x_ref, wt_ref, emb_ref, o_ref):
    xb = x_ref[...]
    lin = jax.lax.dot_general(
        xb, wt_ref[...], (((1,), (0,)), ((), ())),
        preferred_element_type=jnp.float32)
    cat = xb[:, 0:1].astype(jnp.int32)
    iota = jax.lax.broadcasted_iota(jnp.int32, (1, _C + 1), 1)
    onehot = jnp.where(iota == _C, 1.0,
                       (cat == iota).astype(jnp.float32))
    emb = jax.lax.dot_general(
        onehot, emb_ref[...], (((1,), (0,)), ((), ())),
        preferred_element_type=jnp.float32)
    o_ref[...] = lin + emb


def kernel(x, emb_table, W, b):
    wt_pad = jnp.concatenate([jnp.zeros((1, _E), jnp.float32), W.T], axis=0)
    emb2 = jnp.concatenate([emb_table, b.reshape(1, _E)], axis=0)
    return pl.pallas_call(
        _body,
        grid=(_B // _BS,),
        in_specs=[
            pl.BlockSpec((_BS, _D + 1), lambda i: (i, 0)),
            pl.BlockSpec((_D + 1, _E), lambda i: (0, 0)),
            pl.BlockSpec((_C + 1, _E), lambda i: (0, 0)),
        ],
        out_specs=pl.BlockSpec((_BS, _E), lambda i: (i, 0)),
        out_shape=jax.ShapeDtypeStruct((_B, _E), jnp.float32),
    )(x, wt_pad, emb2)
